# baseline (speedup 1.0000x reference)
import functools

import jax
import jax.numpy as jnp
from jax import lax
from jax.experimental import pallas as pl
from jax.experimental.pallas import tpu as pltpu
from jax.experimental.pallas import tpu_sc as plsc

N_USERS = 1000000
N_ITEMS = 100000
D = 64
B = 16384

NC = 2
NS = 16
NW = NC * NS
B_PER_W = B // NW
QB = 128
N_Q = B_PER_W // QB

UB_FULL = N_USERS // 128
IB_FULL = N_ITEMS // 128
U_PAIR_FULL = UB_FULL * 64
I_PAIR_FULL = IB_FULL * 64
U_TAIL_PAIRS = (N_USERS - UB_FULL * 128) // 2
I_TAIL_PAIRS = (N_ITEMS - IB_FULL * 128) // 2


def _transpose_block(blk, out_v, iota16):
    for j in range(64):
        for cc in range(D // 16):
            d16 = iota16 + cc * 16
            v0 = plsc.load_gather(blk, [d16, jnp.full((16,), 2 * j, jnp.int32)])
            v1 = plsc.load_gather(blk, [d16, jnp.full((16,), 2 * j + 1, jnp.int32)])
            out_v[j, pl.ds(cc * 16, 16)] = v0
            out_v[j, pl.ds(D + cc * 16, 16)] = v1


def _stream_transpose(src_hbm, dst_hbm, start, count, max_count,
                      blk0, blk1, out0, out1, sem_i, sem_o, iota16):
    blks = (blk0, blk1)
    outs = (out0, out1)

    @pl.when(count > 0)
    def _():
        col0 = pl.multiple_of(start * 128, 128)
        pltpu.async_copy(src_hbm.at[:, pl.ds(col0, 128)], blk0, sem_i)

    def body(t, carry):
        for b_par in range(2):
            t_b = t * 2 + b_par
            blk = blks[b_par]
            out_v = outs[b_par]
            nblk = blks[(b_par + 1) % 2]

            @pl.when(t_b < count)
            def _():
                @pl.when(t_b + 1 < count)
                def _():
                    ncol = pl.multiple_of((start + t_b + 1) * 128, 128)
                    pltpu.async_copy(src_hbm.at[:, pl.ds(ncol, 128)],
                                     nblk, sem_i)
                pltpu.make_async_copy(src_hbm.at[:, pl.ds(0, 128)],
                                      blk, sem_i).wait()
                @pl.when(t_b >= 2)
                def _():
                    pltpu.make_async_copy(out_v, dst_hbm.at[pl.ds(0, 64), :],
                                          sem_o).wait()
                _transpose_block(blk, out_v, iota16)
                row = pl.multiple_of((start + t_b) * 64, 64)
                pltpu.async_copy(out_v, dst_hbm.at[pl.ds(row, 64), :], sem_o)
        return carry

    lax.fori_loop(0, (max_count + 1) // 2, body, 0)

    for k in range(2):
        @pl.when(count > k)
        def _():
            pltpu.make_async_copy(outs[0], dst_hbm.at[pl.ds(0, 64), :],
                                  sem_o).wait()


def _fmt_kernel(ut_hbm, it_hbm, ut2_hbm, it2_hbm,
                blk0, blk1, out0, out1, sem_i, sem_o):
    wid = lax.axis_index("s") * NC + lax.axis_index("c")
    iota16 = lax.iota(jnp.int32, 16)

    u_start = wid * 244 + jnp.minimum(wid, 4)
    u_count = 244 + jnp.where(wid < 4, 1, 0)
    _stream_transpose(ut_hbm, ut2_hbm, u_start, u_count, 245,
                      blk0, blk1, out0, out1, sem_i, sem_o, iota16)

    i_start = wid * 24 + jnp.minimum(wid, 13)
    i_count = 24 + jnp.where(wid < 13, 1, 0)
    _stream_transpose(it_hbm, it2_hbm, i_start, i_count, 25,
                      blk0, blk1, out0, out1, sem_i, sem_o, iota16)


def _mf_kernel(u2_hbm, u2c_hbm, i2_hbm, i2c_hbm, uh_hbm, ih_hbm,
               ut_hbm, it_hbm, tu_hbm, ti_hbm,
               pred_hbm, p_hbm, q_hbm,
               idx_u, idx_uc, idx_i, idx_ic, uh_v, ih_v,
               tu_v, ti_v, p_big, q_big, p_pair, q_pair,
               pred_v, sem_u, sem_i):
    wid = lax.axis_index("s") * NC + lax.axis_index("c")
    base = wid * B_PER_W
    lanes = lax.iota(jnp.int32, 16)

    pltpu.sync_copy(u2_hbm.at[pl.ds(base, B_PER_W)], idx_u)
    pltpu.sync_copy(u2c_hbm.at[pl.ds(base, B_PER_W)], idx_uc)
    pltpu.sync_copy(i2_hbm.at[pl.ds(base, B_PER_W)], idx_i)
    pltpu.sync_copy(i2c_hbm.at[pl.ds(base, B_PER_W)], idx_ic)
    pltpu.sync_copy(uh_hbm.at[pl.ds(base, B_PER_W)], uh_v)
    pltpu.sync_copy(ih_hbm.at[pl.ds(base, B_PER_W)], ih_v)
    pltpu.sync_copy(tu_hbm, tu_v)
    pltpu.sync_copy(ti_hbm, ti_v)

    for t in range(N_Q):
        cu = pltpu.async_copy(
            ut_hbm.at[idx_uc.at[pl.ds(t * QB, QB)]], p_big, sem_u)
        ci = pltpu.async_copy(
            it_hbm.at[idx_ic.at[pl.ds(t * QB, QB)]], q_big, sem_i)
        cu.wait()
        ci.wait()

        def body(g, carry):
            out = jnp.zeros((16,), jnp.float32)
            u16 = idx_u[pl.ds(t * QB + g * 16, 16)]
            i16 = idx_i[pl.ds(t * QB + g * 16, 16)]
            hu16 = uh_v[pl.ds(t * QB + g * 16, 16)]
            hi16 = ih_v[pl.ds(t * QB + g * 16, 16)]

            for r in range(16):
                b = g * 16 + r
                offu = hu16[r] * D
                offi = hi16[r] * D
                pu = u16[r]
                pi = i16[r]
                tails_u = pu >= U_PAIR_FULL
                tails_i = pi >= I_PAIR_FULL
                locu = jnp.clip(pu - U_PAIR_FULL, 0, U_TAIL_PAIRS - 1) * 128
                loci = jnp.clip(pi - I_PAIR_FULL, 0, I_TAIL_PAIRS - 1) * 128
                pr = g * 8 + r // 2
                po = (r % 2) * D
                acc = None
                for c in range(D // 16):
                    pv = p_big[b, pl.ds(offu + c * 16, 16)]
                    qv = q_big[b, pl.ds(offi + c * 16, 16)]
                    pv = jnp.where(tails_u, tu_v[pl.ds(locu + offu + c * 16, 16)], pv)
                    qv = jnp.where(tails_i, ti_v[pl.ds(loci + offi + c * 16, 16)], qv)
                    p_pair[pr, pl.ds(po + c * 16, 16)] = pv
                    q_pair[pr, pl.ds(po + c * 16, 16)] = qv
                    acc = pv * qv if acc is None else acc + pv * qv
                out = jnp.where(lanes == r, jnp.sum(acc), out)
            pred_v[pl.ds(t * QB + g * 16, 16)] = out
            return carry

        lax.fori_loop(0, QB // 16, body, 0)

        pair_base = pl.multiple_of((base + t * QB) // 2, 64)
        pltpu.sync_copy(p_pair, p_hbm.at[pl.ds(pair_base, QB // 2)])
        pltpu.sync_copy(q_pair, q_hbm.at[pl.ds(pair_base, QB // 2)])

    pltpu.sync_copy(pred_v, pred_hbm.at[pl.ds(base, B_PER_W)])


@jax.jit
def _mf(u, i, user_table, item_table):
    mesh = plsc.VectorSubcoreMesh(core_axis_name="c", subcore_axis_name="s")
    params = pltpu.CompilerParams(needs_layout_passes=False)

    fmt = functools.partial(
        pl.kernel,
        out_type=(
            jax.ShapeDtypeStruct((N_USERS // 2, 2 * D), jnp.float32),
            jax.ShapeDtypeStruct((N_ITEMS // 2, 2 * D), jnp.float32),
        ),
        mesh=mesh,
        compiler_params=params,
        scratch_types=[
            pltpu.VMEM((D, 128), jnp.float32),
            pltpu.VMEM((D, 128), jnp.float32),
            pltpu.VMEM((64, 2 * D), jnp.float32),
            pltpu.VMEM((64, 2 * D), jnp.float32),
            pltpu.SemaphoreType.DMA,
            pltpu.SemaphoreType.DMA,
        ],
    )(_fmt_kernel)

    run = functools.partial(
        pl.kernel,
        out_type=(
            jax.ShapeDtypeStruct((B,), jnp.float32),
            jax.ShapeDtypeStruct((B // 2, 2 * D), jnp.float32),
            jax.ShapeDtypeStruct((B // 2, 2 * D), jnp.float32),
        ),
        mesh=mesh,
        compiler_params=params,
        scratch_types=[
            pltpu.VMEM((B_PER_W,), jnp.int32),
            pltpu.VMEM((B_PER_W,), jnp.int32),
            pltpu.VMEM((B_PER_W,), jnp.int32),
            pltpu.VMEM((B_PER_W,), jnp.int32),
            pltpu.VMEM((B_PER_W,), jnp.int32),
            pltpu.VMEM((B_PER_W,), jnp.int32),
            pltpu.VMEM((U_TAIL_PAIRS * 2 * D,), jnp.float32),
            pltpu.VMEM((I_TAIL_PAIRS * 2 * D,), jnp.float32),
            pltpu.VMEM((QB, 2 * D), jnp.float32),
            pltpu.VMEM((QB, 2 * D), jnp.float32),
            pltpu.VMEM((QB // 2, 2 * D), jnp.float32),
            pltpu.VMEM((QB // 2, 2 * D), jnp.float32),
            pltpu.VMEM((B_PER_W,), jnp.float32),
            pltpu.SemaphoreType.DMA,
            pltpu.SemaphoreType.DMA,
        ],
    )(_mf_kernel)

    ut2, it2 = fmt(user_table.T, item_table.T)
    tail_u = user_table[UB_FULL * 128:].reshape(U_TAIL_PAIRS * 2 * D)
    tail_i = item_table[IB_FULL * 128:].reshape(I_TAIL_PAIRS * 2 * D)
    u2 = u >> 1
    i2 = i >> 1
    u2c = jnp.minimum(u2, U_PAIR_FULL - 1)
    i2c = jnp.minimum(i2, I_PAIR_FULL - 1)
    uh = u & 1
    ih = i & 1
    pred, p, q = run(u2, u2c, i2, i2c, uh, ih, ut2, it2, tail_u, tail_i)
    return pred, p.reshape(B, 1, D), q.reshape(B, D, 1)


def kernel(u, i, user_table, item_table):
    return _mf(u, i, user_table, item_table)


# speedup vs baseline: 1.0649x; 1.0649x over previous
import functools

import jax
import jax.numpy as jnp
from jax import lax
from jax.experimental import pallas as pl
from jax.experimental.pallas import tpu as pltpu
from jax.experimental.pallas import tpu_sc as plsc

N_USERS = 1000000
N_ITEMS = 100000
D = 64
B = 16384

NC = 2
NS = 16
NW = NC * NS
B_PER_W = B // NW
QB = 128
N_Q = B_PER_W // QB

UB_FULL = N_USERS // 128
IB_FULL = N_ITEMS // 128
U_PAIR_FULL = UB_FULL * 64
I_PAIR_FULL = IB_FULL * 64
U_TAIL_PAIRS = (N_USERS - UB_FULL * 128) // 2
I_TAIL_PAIRS = (N_ITEMS - IB_FULL * 128) // 2


def _transpose_block(blk, out_v, iota16):
    dvecs = [iota16 + cc * 16 for cc in range(D // 16)]
    for j2 in range(0, 64, 2):
        vals = []
        for j in (j2, j2 + 1):
            cols = (jnp.full((16,), 2 * j, jnp.int32),
                    jnp.full((16,), 2 * j + 1, jnp.int32))
            for h in range(2):
                for cc in range(D // 16):
                    vals.append(
                        (j, h, cc, plsc.load_gather(blk, [dvecs[cc], cols[h]])))
        for j, h, cc, v in vals:
            out_v[j, pl.ds(h * D + cc * 16, 16)] = v


def _stream_transpose(src_hbm, dst_hbm, start, count, max_count,
                      blk0, blk1, out0, out1, sem_i, sem_o, iota16):
    blks = (blk0, blk1)
    outs = (out0, out1)

    @pl.when(count > 0)
    def _():
        col0 = pl.multiple_of(start * 128, 128)
        pltpu.async_copy(src_hbm.at[:, pl.ds(col0, 128)], blk0, sem_i)

    def body(t, carry):
        for b_par in range(2):
            t_b = t * 2 + b_par
            blk = blks[b_par]
            out_v = outs[b_par]
            nblk = blks[(b_par + 1) % 2]

            @pl.when(t_b < count)
            def _():
                @pl.when(t_b + 1 < count)
                def _():
                    ncol = pl.multiple_of((start + t_b + 1) * 128, 128)
                    pltpu.async_copy(src_hbm.at[:, pl.ds(ncol, 128)],
                                     nblk, sem_i)
                pltpu.make_async_copy(src_hbm.at[:, pl.ds(0, 128)],
                                      blk, sem_i).wait()
                @pl.when(t_b >= 2)
                def _():
                    pltpu.make_async_copy(out_v, dst_hbm.at[pl.ds(0, 64), :],
                                          sem_o).wait()
                _transpose_block(blk, out_v, iota16)
                row = pl.multiple_of((start + t_b) * 64, 64)
                pltpu.async_copy(out_v, dst_hbm.at[pl.ds(row, 64), :], sem_o)
        return carry

    lax.fori_loop(0, (max_count + 1) // 2, body, 0)

    for k in range(2):
        @pl.when(count > k)
        def _():
            pltpu.make_async_copy(outs[0], dst_hbm.at[pl.ds(0, 64), :],
                                  sem_o).wait()


def _fmt_kernel(ut_hbm, it_hbm, ut2_hbm, it2_hbm,
                blk0, blk1, out0, out1, sem_i, sem_o):
    wid = lax.axis_index("s") * NC + lax.axis_index("c")
    iota16 = lax.iota(jnp.int32, 16)

    u_start = wid * 244 + jnp.minimum(wid, 4)
    u_count = 244 + jnp.where(wid < 4, 1, 0)
    _stream_transpose(ut_hbm, ut2_hbm, u_start, u_count, 245,
                      blk0, blk1, out0, out1, sem_i, sem_o, iota16)

    i_start = wid * 24 + jnp.minimum(wid, 13)
    i_count = 24 + jnp.where(wid < 13, 1, 0)
    _stream_transpose(it_hbm, it2_hbm, i_start, i_count, 25,
                      blk0, blk1, out0, out1, sem_i, sem_o, iota16)


def _mf_kernel(u2_hbm, u2c_hbm, i2_hbm, i2c_hbm, uh_hbm, ih_hbm,
               ut_hbm, it_hbm, tu_hbm, ti_hbm,
               pred_hbm, p_hbm, q_hbm,
               idx_u, idx_uc, idx_i, idx_ic, uh_v, ih_v,
               tu_v, ti_v, p_big, q_big, p_pair, q_pair,
               pred_v, sem_u, sem_i):
    wid = lax.axis_index("s") * NC + lax.axis_index("c")
    base = wid * B_PER_W
    lanes = lax.iota(jnp.int32, 16)

    pltpu.sync_copy(u2_hbm.at[pl.ds(base, B_PER_W)], idx_u)
    pltpu.sync_copy(u2c_hbm.at[pl.ds(base, B_PER_W)], idx_uc)
    pltpu.sync_copy(i2_hbm.at[pl.ds(base, B_PER_W)], idx_i)
    pltpu.sync_copy(i2c_hbm.at[pl.ds(base, B_PER_W)], idx_ic)
    pltpu.sync_copy(uh_hbm.at[pl.ds(base, B_PER_W)], uh_v)
    pltpu.sync_copy(ih_hbm.at[pl.ds(base, B_PER_W)], ih_v)
    pltpu.sync_copy(tu_hbm, tu_v)
    pltpu.sync_copy(ti_hbm, ti_v)

    for t in range(N_Q):
        cu = pltpu.async_copy(
            ut_hbm.at[idx_uc.at[pl.ds(t * QB, QB)]], p_big, sem_u)
        ci = pltpu.async_copy(
            it_hbm.at[idx_ic.at[pl.ds(t * QB, QB)]], q_big, sem_i)
        cu.wait()
        ci.wait()

        def body(g, carry):
            out = jnp.zeros((16,), jnp.float32)
            u16 = idx_u[pl.ds(t * QB + g * 16, 16)]
            i16 = idx_i[pl.ds(t * QB + g * 16, 16)]
            hu16 = uh_v[pl.ds(t * QB + g * 16, 16)]
            hi16 = ih_v[pl.ds(t * QB + g * 16, 16)]

            for r in range(16):
                b = g * 16 + r
                offu = hu16[r] * D
                offi = hi16[r] * D
                pu = u16[r]
                pi = i16[r]
                tails_u = pu >= U_PAIR_FULL
                tails_i = pi >= I_PAIR_FULL
                locu = jnp.clip(pu - U_PAIR_FULL, 0, U_TAIL_PAIRS - 1) * 128
                loci = jnp.clip(pi - I_PAIR_FULL, 0, I_TAIL_PAIRS - 1) * 128
                pr = g * 8 + r // 2
                po = (r % 2) * D
                acc = None
                for c in range(D // 16):
                    pv = p_big[b, pl.ds(offu + c * 16, 16)]
                    qv = q_big[b, pl.ds(offi + c * 16, 16)]
                    pv = jnp.where(tails_u, tu_v[pl.ds(locu + offu + c * 16, 16)], pv)
                    qv = jnp.where(tails_i, ti_v[pl.ds(loci + offi + c * 16, 16)], qv)
                    p_pair[pr, pl.ds(po + c * 16, 16)] = pv
                    q_pair[pr, pl.ds(po + c * 16, 16)] = qv
                    acc = pv * qv if acc is None else acc + pv * qv
                out = jnp.where(lanes == r, jnp.sum(acc), out)
            pred_v[pl.ds(t * QB + g * 16, 16)] = out
            return carry

        lax.fori_loop(0, QB // 16, body, 0)

        pair_base = pl.multiple_of((base + t * QB) // 2, 64)
        pltpu.sync_copy(p_pair, p_hbm.at[pl.ds(pair_base, QB // 2)])
        pltpu.sync_copy(q_pair, q_hbm.at[pl.ds(pair_base, QB // 2)])

    pltpu.sync_copy(pred_v, pred_hbm.at[pl.ds(base, B_PER_W)])


@jax.jit
def _mf(u, i, user_table, item_table):
    mesh = plsc.VectorSubcoreMesh(core_axis_name="c", subcore_axis_name="s")
    params = pltpu.CompilerParams(needs_layout_passes=False)

    fmt = functools.partial(
        pl.kernel,
        out_type=(
            jax.ShapeDtypeStruct((N_USERS // 2, 2 * D), jnp.float32),
            jax.ShapeDtypeStruct((N_ITEMS // 2, 2 * D), jnp.float32),
        ),
        mesh=mesh,
        compiler_params=params,
        scratch_types=[
            pltpu.VMEM((D, 128), jnp.float32),
            pltpu.VMEM((D, 128), jnp.float32),
            pltpu.VMEM((64, 2 * D), jnp.float32),
            pltpu.VMEM((64, 2 * D), jnp.float32),
            pltpu.SemaphoreType.DMA,
            pltpu.SemaphoreType.DMA,
        ],
    )(_fmt_kernel)

    run = functools.partial(
        pl.kernel,
        out_type=(
            jax.ShapeDtypeStruct((B,), jnp.float32),
            jax.ShapeDtypeStruct((B // 2, 2 * D), jnp.float32),
            jax.ShapeDtypeStruct((B // 2, 2 * D), jnp.float32),
        ),
        mesh=mesh,
        compiler_params=params,
        scratch_types=[
            pltpu.VMEM((B_PER_W,), jnp.int32),
            pltpu.VMEM((B_PER_W,), jnp.int32),
            pltpu.VMEM((B_PER_W,), jnp.int32),
            pltpu.VMEM((B_PER_W,), jnp.int32),
            pltpu.VMEM((B_PER_W,), jnp.int32),
            pltpu.VMEM((B_PER_W,), jnp.int32),
            pltpu.VMEM((U_TAIL_PAIRS * 2 * D,), jnp.float32),
            pltpu.VMEM((I_TAIL_PAIRS * 2 * D,), jnp.float32),
            pltpu.VMEM((QB, 2 * D), jnp.float32),
            pltpu.VMEM((QB, 2 * D), jnp.float32),
            pltpu.VMEM((QB // 2, 2 * D), jnp.float32),
            pltpu.VMEM((QB // 2, 2 * D), jnp.float32),
            pltpu.VMEM((B_PER_W,), jnp.float32),
            pltpu.SemaphoreType.DMA,
            pltpu.SemaphoreType.DMA,
        ],
    )(_mf_kernel)

    ut2, it2 = fmt(user_table.T, item_table.T)
    tail_u = user_table[UB_FULL * 128:].reshape(U_TAIL_PAIRS * 2 * D)
    tail_i = item_table[IB_FULL * 128:].reshape(I_TAIL_PAIRS * 2 * D)
    u2 = u >> 1
    i2 = i >> 1
    u2c = jnp.minimum(u2, U_PAIR_FULL - 1)
    i2c = jnp.minimum(i2, I_PAIR_FULL - 1)
    uh = u & 1
    ih = i & 1
    pred, p, q = run(u2, u2c, i2, i2c, uh, ih, ut2, it2, tail_u, tail_i)
    return pred, p.reshape(B, 1, D), q.reshape(B, D, 1)


def kernel(u, i, user_table, item_table):
    return _mf(u, i, user_table, item_table)


# speedup vs baseline: 2.1165x; 1.9875x over previous
import functools

import jax
import jax.numpy as jnp
from jax import lax
from jax.experimental import pallas as pl
from jax.experimental.pallas import tpu as pltpu
from jax.experimental.pallas import tpu_sc as plsc

N_USERS = 1000000
N_ITEMS = 100000
D = 64
B = 16384

NC = 2
NS = 16
NW = NC * NS
B_PER_W = B // NW
QB = 128
N_Q = B_PER_W // QB


def _mf_kernel(u_hbm, i_hbm, ut_hbm, it_hbm,
               pred_hbm, p_hbm, q_hbm,
               idx_u, idx_i, p_big, q_big, p_pair, q_pair,
               pred_v, sem_u, sem_i):
    wid = lax.axis_index("s") * NC + lax.axis_index("c")
    base = wid * B_PER_W
    lanes = lax.iota(jnp.int32, 16)

    pltpu.sync_copy(u_hbm.at[pl.ds(base, B_PER_W)], idx_u)
    pltpu.sync_copy(i_hbm.at[pl.ds(base, B_PER_W)], idx_i)

    for t in range(N_Q):
        cu = pltpu.async_copy(
            ut_hbm.at[idx_u.at[pl.ds(t * QB, QB)]], p_big, sem_u)
        ci = pltpu.async_copy(
            it_hbm.at[idx_i.at[pl.ds(t * QB, QB)]], q_big, sem_i)
        cu.wait()
        ci.wait()

        def body(g, carry):
            out = jnp.zeros((16,), jnp.float32)
            for r in range(16):
                b = g * 16 + r
                pr = g * 8 + r // 2
                po = (r % 2) * D
                acc = None
                for c in range(D // 16):
                    pv = p_big[b, pl.ds(c * 16, 16)]
                    qv = q_big[b, pl.ds(c * 16, 16)]
                    p_pair[pr, pl.ds(po + c * 16, 16)] = pv
                    q_pair[pr, pl.ds(po + c * 16, 16)] = qv
                    acc = pv * qv if acc is None else acc + pv * qv
                out = jnp.where(lanes == r, jnp.sum(acc), out)
            pred_v[pl.ds(t * QB + g * 16, 16)] = out
            return carry

        lax.fori_loop(0, QB // 16, body, 0)

        pair_base = pl.multiple_of((base + t * QB) // 2, 64)
        pltpu.sync_copy(p_pair, p_hbm.at[pl.ds(pair_base, QB // 2)])
        pltpu.sync_copy(q_pair, q_hbm.at[pl.ds(pair_base, QB // 2)])

    pltpu.sync_copy(pred_v, pred_hbm.at[pl.ds(base, B_PER_W)])


@jax.jit
def _mf(u, i, user_table, item_table):
    mesh = plsc.VectorSubcoreMesh(core_axis_name="c", subcore_axis_name="s")
    run = functools.partial(
        pl.kernel,
        out_type=(
            jax.ShapeDtypeStruct((B,), jnp.float32),
            jax.ShapeDtypeStruct((B // 2, 2 * D), jnp.float32),
            jax.ShapeDtypeStruct((B // 2, 2 * D), jnp.float32),
        ),
        mesh=mesh,
        compiler_params=pltpu.CompilerParams(needs_layout_passes=False),
        scratch_types=[
            pltpu.VMEM((B_PER_W,), jnp.int32),
            pltpu.VMEM((B_PER_W,), jnp.int32),
            pltpu.VMEM((QB, 2 * D), jnp.float32),
            pltpu.VMEM((QB, 2 * D), jnp.float32),
            pltpu.VMEM((QB // 2, 2 * D), jnp.float32),
            pltpu.VMEM((QB // 2, 2 * D), jnp.float32),
            pltpu.VMEM((B_PER_W,), jnp.float32),
            pltpu.SemaphoreType.DMA,
            pltpu.SemaphoreType.DMA,
        ],
    )(_mf_kernel)
    ut_w = jnp.pad(user_table, ((0, 0), (0, 2 * D - D)))
    it_w = jnp.pad(item_table, ((0, 0), (0, 2 * D - D)))
    pred, p, q = run(u, i, ut_w, it_w)
    return pred, p.reshape(B, 1, D), q.reshape(B, D, 1)


def kernel(u, i, user_table, item_table):
    return _mf(u, i, user_table, item_table)
